# baseline (speedup 1.0000x reference)
import jax
import jax.numpy as jnp
from jax import lax
from jax.experimental import pallas as pl
from jax.experimental.pallas import tpu as pltpu
from jax.experimental.pallas import tpu_sc as plsc

NUM_CODES = 1024
EMB_DIM = 256
NUM_HEADS = 4
HEAD_DIM = EMB_DIM // NUM_HEADS
COMMITMENT_COST = 0.1

BATCH = 65536
ROWS_PER_BLOCK = 1024
HEADS_PAD = 8

SC_CORES = 2
SC_SUBCORES = 16
SC_WORKERS = SC_CORES * SC_SUBCORES
SC_TOKEN_BLOCKS = SC_WORKERS // NUM_HEADS
TOK_PER_TILE = BATCH // SC_TOKEN_BLOCKS
SC_CHUNK = 512
SC_NCHUNK = TOK_PER_TILE // SC_CHUNK
HIST_W = 16
NUM_GIDS = NUM_HEADS * NUM_CODES


def _tc_body(z_ref, znT_ref, w2_ref, wnT_ref,
             idxT_ref, gidxT_ref, loss_ref, loss_acc):
    step = pl.program_id(0)
    nsteps = pl.num_programs(0)

    @pl.when(step == 0)
    def _init():
        loss_acc[0] = jnp.float32(0.0)

    total = jnp.float32(0.0)
    iota_f = lax.broadcasted_iota(
        jnp.int32, (NUM_CODES, ROWS_PER_BLOCK), 0).astype(jnp.float32)
    for h in range(NUM_HEADS):
        zh = z_ref[:, h * HEAD_DIM:(h + 1) * HEAD_DIM]
        w2 = w2_ref[h]
        mm2 = lax.dot_general(w2, zh, (((1,), (1,)), ((), ())),
                              preferred_element_type=jnp.float32)
        zn = znT_ref[h:h + 1, :]
        wn = wnT_ref[:, h:h + 1]
        sq = (zn + wn) + mm2
        x = jnp.maximum(sq, 0.0)
        dist = jnp.sqrt(x)
        m = jnp.min(dist, axis=0)
        idxf = jnp.min(jnp.where(dist == m[None, :], iota_f,
                                 jnp.float32(NUM_CODES)), axis=0)
        idxi = idxf.astype(jnp.int32)
        idxT_ref[h, :] = idxi
        gidxT_ref[h, :] = idxi + jnp.int32(h * NUM_CODES)
        total = total + jnp.sum(m * m)

    zero_row = jnp.zeros((ROWS_PER_BLOCK,), jnp.int32)
    for h in range(NUM_HEADS, HEADS_PAD):
        idxT_ref[h, :] = zero_row
        gidxT_ref[h, :] = zero_row
    loss_acc[0] = loss_acc[0] + total

    @pl.when(step == nsteps - 1)
    def _fin():
        loss_ref[0] = loss_acc[0] * jnp.float32(1.0 / (BATCH * EMB_DIM))


def _tc_stage(z_e, znT, w2m, wnT):
    nblocks = BATCH // ROWS_PER_BLOCK
    return pl.pallas_call(
        _tc_body,
        grid=(nblocks,),
        in_specs=[
            pl.BlockSpec((ROWS_PER_BLOCK, EMB_DIM), lambda i: (i, 0)),
            pl.BlockSpec((HEADS_PAD, ROWS_PER_BLOCK), lambda i: (0, i)),
            pl.BlockSpec((NUM_HEADS, NUM_CODES, HEAD_DIM),
                         lambda i: (0, 0, 0)),
            pl.BlockSpec((NUM_CODES, NUM_HEADS), lambda i: (0, 0)),
        ],
        out_specs=[
            pl.BlockSpec((HEADS_PAD, ROWS_PER_BLOCK), lambda i: (0, i)),
            pl.BlockSpec((HEADS_PAD, ROWS_PER_BLOCK), lambda i: (0, i)),
            pl.BlockSpec(memory_space=pltpu.SMEM),
        ],
        out_shape=[
            jax.ShapeDtypeStruct((HEADS_PAD, BATCH), jnp.int32),
            jax.ShapeDtypeStruct((HEADS_PAD, BATCH), jnp.int32),
            jax.ShapeDtypeStruct((1,), jnp.float32),
        ],
        scratch_shapes=[
            pltpu.SMEM((1,), jnp.float32),
        ],
        compiler_params=pltpu.CompilerParams(
            dimension_semantics=("arbitrary",),
        ),
    )(z_e, znT, w2m, wnT)


def _sc_body(wflat_hbm, gidxT_hbm, zeros_hbm, ones_hbm,
             out_hbm, hist_hbm,
             idx_v0, idx_v1, rows_v0, rows_v1, ones_v, hist_sh,
             sem0, sem1):
    cid = lax.axis_index("c")
    sid = lax.axis_index("s")
    wid = sid * SC_CORES + cid
    head = wid % NUM_HEADS
    tok0 = (wid // NUM_HEADS) * TOK_PER_TILE

    idxs = [idx_v0, idx_v1]
    rows = [rows_v0, rows_v1]
    sems = [sem0, sem1]

    pltpu.sync_copy(ones_hbm, ones_v)

    @pl.when(sid == 0)
    def _init_hist():
        pltpu.sync_copy(zeros_hbm, hist_sh)

    plsc.subcore_barrier()

    pltpu.sync_copy(gidxT_hbm.at[head, pl.ds(tok0, SC_CHUNK)], idxs[0])
    copies = [None] * SC_NCHUNK
    copies[0] = pltpu.async_copy(wflat_hbm.at[idxs[0]], rows[0], sems[0])
    for i in range(SC_NCHUNK):
        cur = i % 2
        if i + 1 < SC_NCHUNK:
            nxt = (i + 1) % 2
            pltpu.sync_copy(
                gidxT_hbm.at[head,
                             pl.ds(tok0 + (i + 1) * SC_CHUNK, SC_CHUNK)],
                idxs[nxt])
            copies[i + 1] = pltpu.async_copy(
                wflat_hbm.at[idxs[nxt]], rows[nxt], sems[nxt])
        copies[i].wait()
        pltpu.sync_copy(
            rows[cur],
            out_hbm.at[pl.ds(tok0 + i * SC_CHUNK, SC_CHUNK),
                       pl.ds(head * HEAD_DIM, HEAD_DIM)])
        pltpu.sync_copy(ones_v, hist_sh.at[idxs[cur]], add=True)

    plsc.subcore_barrier()

    @pl.when(sid == 0)
    def _emit_hist():
        pltpu.sync_copy(hist_sh, hist_hbm.at[cid])


def _sc_stage(wflat, gidxT, zeros_h, ones_h):
    run = pl.kernel(
        _sc_body,
        out_type=[
            jax.ShapeDtypeStruct((BATCH, EMB_DIM), jnp.float32),
            jax.ShapeDtypeStruct((SC_CORES, NUM_GIDS, HIST_W), jnp.float32),
        ],
        mesh=plsc.VectorSubcoreMesh(core_axis_name="c", subcore_axis_name="s"),
        scratch_types=[
            pltpu.VMEM((SC_CHUNK,), jnp.int32),
            pltpu.VMEM((SC_CHUNK,), jnp.int32),
            pltpu.VMEM((SC_CHUNK, HEAD_DIM), jnp.float32),
            pltpu.VMEM((SC_CHUNK, HEAD_DIM), jnp.float32),
            pltpu.VMEM((SC_CHUNK, HIST_W), jnp.float32),
            pltpu.VMEM_SHARED((NUM_GIDS, HIST_W), jnp.float32),
            pltpu.SemaphoreType.DMA,
            pltpu.SemaphoreType.DMA,
        ],
        compiler_params=pltpu.CompilerParams(use_tc_tiling_on_sc=False),
    )
    return run(wflat, gidxT, zeros_h, ones_h)


def _perp_body(hist_ref, perp_ref):
    h = hist_ref[...]
    c = h[0] + h[1]
    s = jnp.sum(c, axis=2)
    avg = s * jnp.float32(1.0 / (HIST_W * BATCH))
    ent = -jnp.sum(avg * jnp.log(avg + 1e-10), axis=1)
    perp_ref[0] = jnp.mean(jnp.exp(ent))


def _perp_stage(hist4):
    return pl.pallas_call(
        _perp_body,
        grid=(1,),
        in_specs=[
            pl.BlockSpec((SC_CORES, NUM_HEADS, NUM_CODES, HIST_W),
                         lambda i: (0, 0, 0, 0)),
        ],
        out_specs=pl.BlockSpec(memory_space=pltpu.SMEM),
        out_shape=jax.ShapeDtypeStruct((1,), jnp.float32),
    )(hist4)


def kernel(z_e, emb_weights):
    zs = z_e.reshape(BATCH, NUM_HEADS, HEAD_DIM)
    zn = jnp.sum(zs * zs, axis=2)
    znT = jnp.pad(zn.T, ((0, HEADS_PAD - NUM_HEADS), (0, 0)))
    w2m = emb_weights * jnp.float32(-2.0)
    wn = jnp.sum(emb_weights * emb_weights, axis=2)
    wnT = wn.T

    idxT, gidxT, loss1 = _tc_stage(z_e, znT, w2m, wnT)

    wflat = emb_weights.reshape(NUM_GIDS, HEAD_DIM)
    zeros_h = jnp.zeros((NUM_GIDS, HIST_W), jnp.float32)
    ones_h = jnp.ones((SC_CHUNK, HIST_W), jnp.float32)
    z_q, hist = _sc_stage(wflat, gidxT, zeros_h, ones_h)

    perplexity = _perp_stage(
        hist.reshape(SC_CORES, NUM_HEADS, NUM_CODES, HIST_W))[0]

    idx = idxT[:NUM_HEADS].T
    codebook_loss = loss1[0]
    commitment_loss = jnp.float32(COMMITMENT_COST) * codebook_loss
    return (z_q, idx, codebook_loss, commitment_loss, perplexity)


# speedup vs baseline: 1.0896x; 1.0896x over previous
import jax
import jax.numpy as jnp
from jax import lax
from jax.experimental import pallas as pl
from jax.experimental.pallas import tpu as pltpu
from jax.experimental.pallas import tpu_sc as plsc

NUM_CODES = 1024
EMB_DIM = 256
NUM_HEADS = 4
HEAD_DIM = EMB_DIM // NUM_HEADS
COMMITMENT_COST = 0.1

BATCH = 65536
ROWS_PER_BLOCK = 1024
HEADS_PAD = 8

SC_CORES = 2
SC_SUBCORES = 16
SC_WORKERS = SC_CORES * SC_SUBCORES
SC_TOKEN_BLOCKS = SC_WORKERS // NUM_HEADS
TOK_PER_TILE = BATCH // SC_TOKEN_BLOCKS
SC_CHUNK = 512
SC_NCHUNK = TOK_PER_TILE // SC_CHUNK
HIST_W = 16
NUM_GIDS = NUM_HEADS * NUM_CODES


def _tc_body(z_ref, znT_ref, w2_ref, wnT_ref,
             idxT_ref, gidxT_ref, loss_ref, loss_acc):
    step = pl.program_id(0)
    nsteps = pl.num_programs(0)

    @pl.when(step == 0)
    def _init():
        loss_acc[0] = jnp.float32(0.0)

    total = jnp.float32(0.0)
    iota_f = lax.broadcasted_iota(
        jnp.int32, (NUM_CODES, ROWS_PER_BLOCK), 0).astype(jnp.float32)
    for h in range(NUM_HEADS):
        zh = z_ref[:, h * HEAD_DIM:(h + 1) * HEAD_DIM]
        w2 = w2_ref[h]
        mm2 = lax.dot_general(w2, zh, (((1,), (1,)), ((), ())),
                              preferred_element_type=jnp.float32)
        zn = znT_ref[h:h + 1, :]
        wn = wnT_ref[:, h:h + 1]
        sq = (zn + wn) + mm2
        x = jnp.maximum(sq, 0.0)
        dist = jnp.sqrt(x)
        m = jnp.min(dist, axis=0)
        idxf = jnp.min(jnp.where(dist == m[None, :], iota_f,
                                 jnp.float32(NUM_CODES)), axis=0)
        idxi = idxf.astype(jnp.int32)
        idxT_ref[h, :] = idxi
        gidxT_ref[h, :] = idxi + jnp.int32(h * NUM_CODES)
        total = total + jnp.sum(m * m)

    zero_row = jnp.zeros((ROWS_PER_BLOCK,), jnp.int32)
    for h in range(NUM_HEADS, HEADS_PAD):
        idxT_ref[h, :] = zero_row
        gidxT_ref[h, :] = zero_row
    loss_acc[0] = loss_acc[0] + total

    @pl.when(step == nsteps - 1)
    def _fin():
        loss_ref[0] = loss_acc[0] * jnp.float32(1.0 / (BATCH * EMB_DIM))


def _tc_stage(z_e, znT, w2m, wnT):
    nblocks = BATCH // ROWS_PER_BLOCK
    return pl.pallas_call(
        _tc_body,
        grid=(nblocks,),
        in_specs=[
            pl.BlockSpec((ROWS_PER_BLOCK, EMB_DIM), lambda i: (i, 0)),
            pl.BlockSpec((HEADS_PAD, ROWS_PER_BLOCK), lambda i: (0, i)),
            pl.BlockSpec((NUM_HEADS, NUM_CODES, HEAD_DIM),
                         lambda i: (0, 0, 0)),
            pl.BlockSpec((NUM_CODES, NUM_HEADS), lambda i: (0, 0)),
        ],
        out_specs=[
            pl.BlockSpec((HEADS_PAD, ROWS_PER_BLOCK), lambda i: (0, i)),
            pl.BlockSpec((HEADS_PAD, ROWS_PER_BLOCK), lambda i: (0, i)),
            pl.BlockSpec(memory_space=pltpu.SMEM),
        ],
        out_shape=[
            jax.ShapeDtypeStruct((HEADS_PAD, BATCH), jnp.int32),
            jax.ShapeDtypeStruct((HEADS_PAD, BATCH), jnp.int32),
            jax.ShapeDtypeStruct((1,), jnp.float32),
        ],
        scratch_shapes=[
            pltpu.SMEM((1,), jnp.float32),
        ],
        compiler_params=pltpu.CompilerParams(
            dimension_semantics=("arbitrary",),
        ),
    )(z_e, znT, w2m, wnT)


def _sc_body(wflat_hbm, gidxT_hbm, zeros_hbm, ones_hbm,
             out_hbm, hist_hbm,
             idx_v0, idx_v1, rows_v0, rows_v1, ones_v, hist_sh,
             sem0, sem1):
    cid = lax.axis_index("c")
    sid = lax.axis_index("s")
    wid = sid * SC_CORES + cid
    head = wid % NUM_HEADS
    tok0 = (wid // NUM_HEADS) * TOK_PER_TILE

    idxs = [idx_v0, idx_v1]
    rows = [rows_v0, rows_v1]
    sems = [sem0, sem1]

    pltpu.sync_copy(ones_hbm, ones_v)

    @pl.when(sid == 0)
    def _init_hist():
        pltpu.sync_copy(zeros_hbm, hist_sh)

    plsc.subcore_barrier()

    pltpu.sync_copy(gidxT_hbm.at[head, pl.ds(tok0, SC_CHUNK)], idxs[0])
    copies = [None] * SC_NCHUNK
    copies[0] = pltpu.async_copy(wflat_hbm.at[idxs[0]], rows[0], sems[0])
    for i in range(SC_NCHUNK):
        cur = i % 2
        if i + 1 < SC_NCHUNK:
            nxt = (i + 1) % 2
            pltpu.sync_copy(
                gidxT_hbm.at[head,
                             pl.ds(tok0 + (i + 1) * SC_CHUNK, SC_CHUNK)],
                idxs[nxt])
            copies[i + 1] = pltpu.async_copy(
                wflat_hbm.at[idxs[nxt]], rows[nxt], sems[nxt])
        copies[i].wait()
        pltpu.sync_copy(
            rows[cur],
            out_hbm.at[pl.ds(tok0 + i * SC_CHUNK, SC_CHUNK),
                       pl.ds(head * HEAD_DIM, HEAD_DIM)])
        pltpu.sync_copy(ones_v, hist_sh.at[idxs[cur]], add=True)

    plsc.subcore_barrier()

    @pl.when(sid == 0)
    def _emit_hist():
        pltpu.sync_copy(hist_sh, hist_hbm.at[cid])


def _sc_stage(wflat, gidxT, zeros_h, ones_h):
    run = pl.kernel(
        _sc_body,
        out_type=[
            jax.ShapeDtypeStruct((BATCH, EMB_DIM), jnp.float32),
            jax.ShapeDtypeStruct((SC_CORES, NUM_GIDS, HIST_W), jnp.float32),
        ],
        mesh=plsc.VectorSubcoreMesh(core_axis_name="c", subcore_axis_name="s"),
        scratch_types=[
            pltpu.VMEM((SC_CHUNK,), jnp.int32),
            pltpu.VMEM((SC_CHUNK,), jnp.int32),
            pltpu.VMEM((SC_CHUNK, HEAD_DIM), jnp.float32),
            pltpu.VMEM((SC_CHUNK, HEAD_DIM), jnp.float32),
            pltpu.VMEM((SC_CHUNK, HIST_W), jnp.float32),
            pltpu.VMEM_SHARED((NUM_GIDS, HIST_W), jnp.float32),
            pltpu.SemaphoreType.DMA,
            pltpu.SemaphoreType.DMA,
        ],
        compiler_params=pltpu.CompilerParams(use_tc_tiling_on_sc=False),
    )
    return run(wflat, gidxT, zeros_h, ones_h)


def _perp_body(hist_ref, perp_ref):
    h = hist_ref[...]
    c = h[0] + h[1]
    s = jnp.sum(c, axis=2)
    avg = s * jnp.float32(1.0 / (HIST_W * BATCH))
    ent = -jnp.sum(avg * jnp.log(avg + 1e-10), axis=1)
    perp_ref[0] = jnp.mean(jnp.exp(ent))


def _perp_stage(hist4):
    return pl.pallas_call(
        _perp_body,
        grid=(1,),
        in_specs=[
            pl.BlockSpec((SC_CORES, NUM_HEADS, NUM_CODES, HIST_W),
                         lambda i: (0, 0, 0, 0)),
        ],
        out_specs=pl.BlockSpec(memory_space=pltpu.SMEM),
        out_shape=jax.ShapeDtypeStruct((1,), jnp.float32),
    )(hist4)


def kernel(z_e, emb_weights):
    zn_rows = [
        jnp.sum(z_e[:, h * HEAD_DIM:(h + 1) * HEAD_DIM]
                * z_e[:, h * HEAD_DIM:(h + 1) * HEAD_DIM], axis=1)
        for h in range(NUM_HEADS)
    ]
    znT = jnp.pad(jnp.stack(zn_rows, axis=0),
                  ((0, HEADS_PAD - NUM_HEADS), (0, 0)))
    w2m = emb_weights * jnp.float32(-2.0)
    wn = jnp.sum(emb_weights * emb_weights, axis=2)
    wnT = wn.T

    idxT, gidxT, loss1 = _tc_stage(z_e, znT, w2m, wnT)

    wflat = emb_weights.reshape(NUM_GIDS, HEAD_DIM)
    zeros_h = jnp.zeros((NUM_GIDS, HIST_W), jnp.float32)
    ones_h = jnp.ones((SC_CHUNK, HIST_W), jnp.float32)
    z_q, hist = _sc_stage(wflat, gidxT, zeros_h, ones_h)

    perplexity = _perp_stage(
        hist.reshape(SC_CORES, NUM_HEADS, NUM_CODES, HIST_W))[0]

    idx = idxT[:NUM_HEADS].T
    codebook_loss = loss1[0]
    commitment_loss = jnp.float32(COMMITMENT_COST) * codebook_loss
    return (z_q, idx, codebook_loss, commitment_loss, perplexity)
